# baseline (speedup 1.0000x reference)
import functools

import jax
import jax.numpy as jnp
from jax import lax
from jax.experimental import pallas as pl
from jax.experimental.pallas import tpu as pltpu
from jax.experimental.pallas import tpu_sc as plsc

B = 32
S = 1024
AUDIO_DIM = 768
VIDEO_DIM = 1024
TEXT_DIM = 768
HIDDEN = 512
N_SEGMENTS = 4
FRAME_RATIO = 60
ALPHAS = [2.0 ** k for k in range(-3, 2)]

SBLK = 1024
NSPLIT = 4
HB = B // NSPLIT
WPB = 32 // HB

_frames_per_segment = S / N_SEGMENTS
_sel_per_segment = max(1, int(_frames_per_segment * FRAME_RATIO / 100))
NSEL = min(_sel_per_segment * N_SEGMENTS, S)

HP = NSEL // WPB
CH = 32
NCH = HP // CH + 1
ALIGNED = (NCH - 1) * CH
TAILSTART = HP - CH
IPADH = (HP + 7) // 8 * 8 + 0
IPADH = ((HP + 15) // 16) * 16

NC = 2
NS = 16
LANES = 16


def _scores_body(a_ref, v_ref, t_ref, wa_ref, wv_ref, wt_ref, bav_ref,
                 bt_ref, o_ref):
    nt = (((1,), (1,)), ((), ()))
    a = a_ref[0]
    v = v_ref[0]
    af = lax.dot_general(a, wa_ref[...], nt,
                         preferred_element_type=jnp.float32)
    vf = lax.dot_general(v, wv_ref[...], nt,
                         preferred_element_type=jnp.float32)
    av = (af + vf) * 0.5 + bav_ref[...]
    tq = lax.dot_general(t_ref[0], wt_ref[...], nt,
                         preferred_element_type=jnp.float32) + bt_ref[...]
    d = av - tq
    d2 = jnp.sum(d * d, axis=1, keepdims=True)
    sc = jnp.zeros_like(d2)
    for alpha in ALPHAS:
        sc = sc + jnp.exp(d2 * (-1.0 / (2.0 * alpha)))
    o_ref[...] = sc.reshape(1, SBLK, 1)


def _scores_call(audio, video, text, wa, wv, wt, bav, bt2, h):
    return pl.pallas_call(
        _scores_body,
        grid=(HB,),
        in_specs=[
            pl.BlockSpec((1, SBLK, AUDIO_DIM), lambda bb: (h * HB + bb, 0, 0)),
            pl.BlockSpec((1, SBLK, VIDEO_DIM), lambda bb: (h * HB + bb, 0, 0)),
            pl.BlockSpec((1, 1, TEXT_DIM), lambda bb: (h * HB + bb, 0, 0)),
            pl.BlockSpec((HIDDEN, AUDIO_DIM), lambda bb: (0, 0)),
            pl.BlockSpec((HIDDEN, VIDEO_DIM), lambda bb: (0, 0)),
            pl.BlockSpec((HIDDEN, TEXT_DIM), lambda bb: (0, 0)),
            pl.BlockSpec((1, HIDDEN), lambda bb: (0, 0)),
            pl.BlockSpec((1, HIDDEN), lambda bb: (0, 0)),
        ],
        out_specs=pl.BlockSpec((1, SBLK, 1), lambda bb: (bb, 0, 0)),
        out_shape=jax.ShapeDtypeStruct((HB, SBLK, 1), jnp.float32),
    )(audio, video, text, wa, wv, wt, bav, bt2)


def _rank_body(s_ref, st_ref, r_ref, p_ref):
    b = pl.program_id(0)
    srow = s_ref[0]
    st = st_ref[...]
    lane = lax.broadcasted_iota(jnp.int32, (S, HB), 1)
    scol = jnp.sum(jnp.where(lane == b, st, 0.0), axis=1, keepdims=True)
    jcol = lax.broadcasted_iota(jnp.int32, (S, 128), 0)
    ones_row = jnp.ones((1, S), jnp.float32)
    sb = jnp.broadcast_to(scol, (S, 128))
    rank_chunks = []
    for ib in range(S // 128):
        chunk = srow[:, ib * 128:(ib + 1) * 128]
        ci = jnp.broadcast_to(chunk, (S, 128))
        ig = ib * 128 + lax.broadcasted_iota(jnp.int32, (S, 128), 1)
        beats = jnp.where(sb > ci, 1.0, 0.0) + \
            jnp.where((sb == ci) & (jcol < ig), 1.0, 0.0)
        rank_chunks.append(jnp.dot(ones_row, beats,
                                   preferred_element_type=jnp.float32))
    rank_row = jnp.concatenate(rank_chunks, axis=1)
    selrow = jnp.where(rank_row < float(NSEL), 1.0, 0.0)
    pos_chunks = []
    for ib in range(S // 128):
        ig = ib * 128 + lax.broadcasted_iota(jnp.int32, (S, 128), 1)
        lt = jnp.where(jcol < ig, 1.0, 0.0)
        pos_chunks.append(jnp.dot(selrow, lt,
                                  preferred_element_type=jnp.float32))
    pos_row = jnp.concatenate(pos_chunks, axis=1)
    r_ref[...] = rank_row.astype(jnp.int32).reshape(1, 8, 128)
    p_ref[...] = pos_row.astype(jnp.int32).reshape(1, 8, 128)


def _rank_call(scores3, scores_t):
    return pl.pallas_call(
        _rank_body,
        grid=(HB,),
        in_specs=[
            pl.BlockSpec((1, 1, S), lambda b: (b, 0, 0)),
            pl.BlockSpec((S, HB), lambda b: (0, 0)),
        ],
        out_specs=[
            pl.BlockSpec((1, 8, 128), lambda b: (b, 0, 0)),
            pl.BlockSpec((1, 8, 128), lambda b: (b, 0, 0)),
        ],
        out_shape=[
            jax.ShapeDtypeStruct((HB, 8, 128), jnp.int32),
            jax.ShapeDtypeStruct((HB, 8, 128), jnp.int32),
        ],
    )(scores3, scores_t)


@functools.lru_cache(maxsize=None)
def _get_sc_kernel(h):
    return _build_sc_kernel(h)


def _build_sc_kernel(h):
  @functools.partial(
    pl.kernel,
    mesh=plsc.VectorSubcoreMesh(core_axis_name="c", subcore_axis_name="s",
                                num_cores=NC, num_subcores=NS),
    compiler_params=pltpu.CompilerParams(needs_layout_passes=False),
    out_type=[],
    scratch_types=[
        pltpu.VMEM((8, 128), jnp.int32),
        pltpu.VMEM((8, 128), jnp.int32),
        pltpu.VMEM((IPADH,), jnp.int32),
        pltpu.VMEM((NCH, CH), jnp.int32),
        pltpu.VMEM((NCH, CH), jnp.int32),
        pltpu.VMEM((2, CH, AUDIO_DIM), jnp.float32),
        pltpu.VMEM((2, CH, VIDEO_DIM), jnp.float32),
        pltpu.SemaphoreType.DMA,
        pltpu.SemaphoreType.DMA,
        pltpu.SemaphoreType.DMA,
        pltpu.SemaphoreType.DMA,
        pltpu.SemaphoreType.DMA,
        pltpu.SemaphoreType.DMA,
        pltpu.SemaphoreType.DMA,
        pltpu.SemaphoreType.DMA,
    ],
  )
  def _sc_select_gather(rank_hbm, pos_hbm, aud_hbm, vid_hbm, aout, vout, iout,
                        rank_v, pos_v, idxl_v, idxg_v, oidx_v, abuf, vbuf,
                        ga0, ga1, gv0, gv1, sa0, sa1, sv0, sv1):
    w = lax.axis_index("s") * NC + lax.axis_index("c")
    bh = lax.shift_right_logical(w, 2)
    ph = lax.bitwise_and(w, 3)
    b = bh + h * HB
    qlo = ph * HP
    pltpu.sync_copy(rank_hbm.at[bh], rank_v)
    pltpu.sync_copy(pos_hbm.at[bh], pos_v)

    base = b * S

    def sel_body(c, carry):
        cr = lax.shift_right_logical(c, 3)
        cl = lax.bitwise_and(c, 7) * LANES
        r = rank_v[cr, pl.ds(cl, LANES)]
        p = pos_v[cr, pl.ds(cl, LANES)]
        q = p - qlo
        inr = (r < NSEL) & (q >= 0) & (q < HP)
        ii = lax.iota(jnp.int32, LANES) + c * LANES
        plsc.store_scatter(idxl_v, [q], ii, mask=inr)
        g = ii + base
        row = lax.shift_right_logical(q, 5)
        col = lax.bitwise_and(q, CH - 1)
        plsc.store_scatter(idxg_v, [row, col], g, mask=inr & (q < ALIGNED))
        rowt = jnp.full((LANES,), NCH - 1, jnp.int32)
        plsc.store_scatter(idxg_v, [rowt, q - TAILSTART], g,
                           mask=inr & (q >= TAILSTART))
        return carry

    lax.fori_loop(0, S // LANES, sel_body, jnp.int32(0))

    def oidx_body(c, carry):
        jq = lax.iota(jnp.int32, LANES) + c * LANES
        dstrow = lax.shift_left(jq + qlo, 5) + b
        row = lax.shift_right_logical(jq, 5)
        col = lax.bitwise_and(jq, CH - 1)
        plsc.store_scatter(oidx_v, [row, col], dstrow, mask=jq < ALIGNED)
        rowt = jnp.full((LANES,), NCH - 1, jnp.int32)
        plsc.store_scatter(oidx_v, [rowt, jq - TAILSTART], dstrow,
                           mask=(jq >= TAILSTART) & (jq < HP))
        return carry

    lax.fori_loop(0, IPADH // LANES, oidx_body, jnp.int32(0))

    pltpu.sync_copy(idxl_v, iout.at[pl.ds((b * WPB + ph) * IPADH, IPADH)])

    gas = (ga0, ga1)
    gvs = (gv0, gv1)
    sas = (sa0, sa1)
    svs = (sv0, sv1)
    sc_a = {}
    sc_v = {}
    ca = pltpu.async_copy(aud_hbm.at[idxg_v.at[0]], abuf.at[0], gas[0])
    cv = pltpu.async_copy(vid_hbm.at[idxg_v.at[0]], vbuf.at[0], gvs[0])
    for k in range(NCH):
        cur, nxt = k % 2, (k + 1) % 2
        if k + 1 < NCH:
            if k >= 1:
                sc_a[k - 1].wait()
                sc_v[k - 1].wait()
            ca_n = pltpu.async_copy(aud_hbm.at[idxg_v.at[k + 1]],
                                    abuf.at[nxt], gas[nxt])
            cv_n = pltpu.async_copy(vid_hbm.at[idxg_v.at[k + 1]],
                                    vbuf.at[nxt], gvs[nxt])
        ca.wait()
        sc_a[k] = pltpu.async_copy(abuf.at[cur], aout.at[oidx_v.at[k]],
                                   sas[cur])
        cv.wait()
        sc_v[k] = pltpu.async_copy(vbuf.at[cur], vout.at[oidx_v.at[k]],
                                   svs[cur])
        if k + 1 < NCH:
            ca, cv = ca_n, cv_n
    sc_a[NCH - 2].wait()
    sc_v[NCH - 2].wait()
    sc_a[NCH - 1].wait()
    sc_v[NCH - 1].wait()

  return _sc_select_gather


def kernel(audio_sequence, video_sequence, text_global, Wa, ba, Wv, bv, Wt, bt):
    bav = ((ba + bv) * 0.5).reshape(1, HIDDEN)
    bt2 = bt.reshape(1, HIDDEN)
    text3 = text_global.reshape(B, 1, TEXT_DIM)
    audio_flat = audio_sequence.reshape(B * S, AUDIO_DIM)
    video_flat = video_sequence.reshape(B * S, VIDEO_DIM)

    aref = jax.new_ref(lax.empty((NSEL * B, AUDIO_DIM), jnp.float32))
    vref = jax.new_ref(lax.empty((NSEL * B, VIDEO_DIM), jnp.float32))
    iref = jax.new_ref(lax.empty((B * WPB * IPADH,), jnp.int32))

    for h in range(NSPLIT):
        scores3 = _scores_call(audio_sequence, video_sequence, text3,
                               Wa, Wv, Wt, bav, bt2, h)
        scores = scores3.reshape(HB, S)
        rank3, pos3 = _rank_call(scores.reshape(HB, 1, S), scores.T)
        _get_sc_kernel(h)(rank3, pos3, audio_flat, video_flat,
                          aref, vref, iref)

    aout = aref[...]
    vout = vref[...]
    iout = iref[...]
    selected_audio = jnp.transpose(
        aout.reshape(NSEL, B, AUDIO_DIM), (1, 0, 2))
    selected_video = jnp.transpose(
        vout.reshape(NSEL, B, VIDEO_DIM), (1, 0, 2))
    ihalves = iout.reshape(B, WPB, IPADH)
    selected_indices = jnp.concatenate(
        [ihalves[:, i, :HP] for i in range(WPB)], axis=1)
    mask = jnp.ones((B, NSEL), jnp.float32)
    return selected_audio, selected_video, selected_indices, mask


# speedup vs baseline: 1.1078x; 1.1078x over previous
import functools

import jax
import jax.numpy as jnp
from jax import lax
from jax.experimental import pallas as pl
from jax.experimental.pallas import tpu as pltpu
from jax.experimental.pallas import tpu_sc as plsc

B = 32
S = 1024
AUDIO_DIM = 768
VIDEO_DIM = 1024
TEXT_DIM = 768
HIDDEN = 512
N_SEGMENTS = 4
FRAME_RATIO = 60
ALPHAS = [2.0 ** k for k in range(-3, 2)]

SBLK = 1024
NSB = S // SBLK

_frames_per_segment = S / N_SEGMENTS
_sel_per_segment = max(1, int(_frames_per_segment * FRAME_RATIO / 100))
NSEL = min(_sel_per_segment * N_SEGMENTS, S)

CH = 32
NCH = 20
LASTSTART = NSEL - CH
ALIGNED = (NCH - 1) * CH
IPAD = 640

NC = 2
NS = 16
LANES = 16


def _scores_body(a_ref, v_ref, t_ref, wa_ref, wv_ref, wt_ref, bav_ref,
                 bt_ref, o_ref):
    nt = (((1,), (1,)), ((), ()))
    a = a_ref[0]
    v = v_ref[0]
    af = lax.dot_general(a, wa_ref[...], nt,
                         preferred_element_type=jnp.float32)
    vf = lax.dot_general(v, wv_ref[...], nt,
                         preferred_element_type=jnp.float32)
    av = (af + vf) * 0.5 + bav_ref[...]
    tq = lax.dot_general(t_ref[0], wt_ref[...], nt,
                         preferred_element_type=jnp.float32) + bt_ref[...]
    d = av - tq
    d2 = jnp.sum(d * d, axis=1, keepdims=True)
    sc = jnp.zeros_like(d2)
    for alpha in ALPHAS:
        sc = sc + jnp.exp(d2 * (-1.0 / (2.0 * alpha)))
    o_ref[...] = sc.reshape(1, SBLK, 1)


def _scores_call(audio, video, text, wa_t, wv_t, wt_t, bav, bt2):
    return pl.pallas_call(
        _scores_body,
        grid=(B, NSB),
        in_specs=[
            pl.BlockSpec((1, SBLK, AUDIO_DIM), lambda b, s: (b, s, 0)),
            pl.BlockSpec((1, SBLK, VIDEO_DIM), lambda b, s: (b, s, 0)),
            pl.BlockSpec((1, 1, TEXT_DIM), lambda b, s: (b, 0, 0)),
            pl.BlockSpec((HIDDEN, AUDIO_DIM), lambda b, s: (0, 0)),
            pl.BlockSpec((HIDDEN, VIDEO_DIM), lambda b, s: (0, 0)),
            pl.BlockSpec((HIDDEN, TEXT_DIM), lambda b, s: (0, 0)),
            pl.BlockSpec((1, HIDDEN), lambda b, s: (0, 0)),
            pl.BlockSpec((1, HIDDEN), lambda b, s: (0, 0)),
        ],
        out_specs=pl.BlockSpec((1, SBLK, 1), lambda b, s: (b * NSB + s, 0, 0)),
        out_shape=jax.ShapeDtypeStruct((B * NSB, SBLK, 1), jnp.float32),
    )(audio, video, text, wa_t, wv_t, wt_t, bav, bt2)


def _rank_body(s_ref, st_ref, r_ref, p_ref):
    b = pl.program_id(0)
    srow = s_ref[0]
    st = st_ref[...]
    lane = lax.broadcasted_iota(jnp.int32, (S, B), 1)
    scol = jnp.sum(jnp.where(lane == b, st, 0.0), axis=1, keepdims=True)
    jcol = lax.broadcasted_iota(jnp.int32, (S, 128), 0)
    ones_row = jnp.ones((1, S), jnp.float32)
    sb = jnp.broadcast_to(scol, (S, 128))
    rank_chunks = []
    for ib in range(S // 128):
        chunk = srow[:, ib * 128:(ib + 1) * 128]
        ci = jnp.broadcast_to(chunk, (S, 128))
        ig = ib * 128 + lax.broadcasted_iota(jnp.int32, (S, 128), 1)
        beats = jnp.where(sb > ci, 1.0, 0.0) + \
            jnp.where((sb == ci) & (jcol < ig), 1.0, 0.0)
        rank_chunks.append(jnp.dot(ones_row, beats,
                                   preferred_element_type=jnp.float32))
    rank_row = jnp.concatenate(rank_chunks, axis=1)
    selrow = jnp.where(rank_row < float(NSEL), 1.0, 0.0)
    pos_chunks = []
    for ib in range(S // 128):
        ig = ib * 128 + lax.broadcasted_iota(jnp.int32, (S, 128), 1)
        lt = jnp.where(jcol < ig, 1.0, 0.0)
        pos_chunks.append(jnp.dot(selrow, lt,
                                  preferred_element_type=jnp.float32))
    pos_row = jnp.concatenate(pos_chunks, axis=1)
    r_ref[...] = rank_row.astype(jnp.int32).reshape(1, 8, 128)
    p_ref[...] = pos_row.astype(jnp.int32).reshape(1, 8, 128)


def _rank_call(scores3, scores_t):
    return pl.pallas_call(
        _rank_body,
        grid=(B,),
        in_specs=[
            pl.BlockSpec((1, 1, S), lambda b: (b, 0, 0)),
            pl.BlockSpec((S, B), lambda b: (0, 0)),
        ],
        out_specs=[
            pl.BlockSpec((1, 8, 128), lambda b: (b, 0, 0)),
            pl.BlockSpec((1, 8, 128), lambda b: (b, 0, 0)),
        ],
        out_shape=[
            jax.ShapeDtypeStruct((B, 8, 128), jnp.int32),
            jax.ShapeDtypeStruct((B, 8, 128), jnp.int32),
        ],
    )(scores3, scores_t)


@functools.lru_cache(maxsize=None)
def _get_sc_kernel():
    return _build_sc_kernel()


def _build_sc_kernel():
  @functools.partial(
    pl.kernel,
    mesh=plsc.VectorSubcoreMesh(core_axis_name="c", subcore_axis_name="s",
                                num_cores=NC, num_subcores=NS),
    compiler_params=pltpu.CompilerParams(needs_layout_passes=False),
    out_type=[
        jax.ShapeDtypeStruct((NSEL * B, AUDIO_DIM), jnp.float32),
        jax.ShapeDtypeStruct((NSEL * B, VIDEO_DIM), jnp.float32),
        jax.ShapeDtypeStruct((B * IPAD,), jnp.int32),
    ],
    scratch_types=[
        pltpu.VMEM((8, 128), jnp.int32),
        pltpu.VMEM((8, 128), jnp.int32),
        pltpu.VMEM((IPAD,), jnp.int32),
        pltpu.VMEM((NCH, CH), jnp.int32),
        pltpu.VMEM((NCH, CH), jnp.int32),
        pltpu.VMEM((2, CH, AUDIO_DIM), jnp.float32),
        pltpu.VMEM((2, CH, VIDEO_DIM), jnp.float32),
        pltpu.SemaphoreType.DMA,
        pltpu.SemaphoreType.DMA,
        pltpu.SemaphoreType.DMA,
        pltpu.SemaphoreType.DMA,
        pltpu.SemaphoreType.DMA,
        pltpu.SemaphoreType.DMA,
        pltpu.SemaphoreType.DMA,
        pltpu.SemaphoreType.DMA,
    ],
  )
  def _sc_select_gather(rank_hbm, pos_hbm, aud_hbm, vid_hbm, aout, vout, iout,
                        rank_v, pos_v, idxl_v, idxg_v, oidx_v, abuf, vbuf,
                        ga0, ga1, gv0, gv1, sa0, sa1, sv0, sv1):
    b = lax.axis_index("s") * NC + lax.axis_index("c")
    pltpu.sync_copy(rank_hbm.at[b], rank_v)
    pltpu.sync_copy(pos_hbm.at[b], pos_v)

    base = b * S

    def sel_body(c, carry):
        cr = lax.shift_right_logical(c, 3)
        cl = lax.bitwise_and(c, 7) * LANES
        r = rank_v[cr, pl.ds(cl, LANES)]
        p = pos_v[cr, pl.ds(cl, LANES)]
        m = r < NSEL
        ii = lax.iota(jnp.int32, LANES) + c * LANES
        plsc.store_scatter(idxl_v, [p], ii, mask=m)
        g = ii + base
        row = lax.shift_right_logical(p, 5)
        col = lax.bitwise_and(p, CH - 1)
        plsc.store_scatter(idxg_v, [row, col], g, mask=m & (p < ALIGNED))
        tail = m & (p >= LASTSTART)
        rowt = jnp.full((LANES,), NCH - 1, jnp.int32)
        plsc.store_scatter(idxg_v, [rowt, p - LASTSTART], g, mask=tail)
        return carry

    lax.fori_loop(0, S // LANES, sel_body, jnp.int32(0))

    def oidx_body(c, carry):
        j = lax.iota(jnp.int32, LANES) + c * LANES
        dstrow = lax.shift_left(j, 5) + b
        row = lax.shift_right_logical(j, 5)
        col = lax.bitwise_and(j, CH - 1)
        plsc.store_scatter(oidx_v, [row, col], dstrow, mask=j < ALIGNED)
        tail = (j >= LASTSTART) & (j < NSEL)
        rowt = jnp.full((LANES,), NCH - 1, jnp.int32)
        plsc.store_scatter(oidx_v, [rowt, j - LASTSTART], dstrow, mask=tail)
        return carry

    lax.fori_loop(0, NSEL // LANES + 1, oidx_body, jnp.int32(0))

    pltpu.sync_copy(idxl_v, iout.at[pl.ds(b * IPAD, IPAD)])

    gas = (ga0, ga1)
    gvs = (gv0, gv1)
    sas = (sa0, sa1)
    svs = (sv0, sv1)
    sc_a = {}
    sc_v = {}
    ca = pltpu.async_copy(aud_hbm.at[idxg_v.at[0]], abuf.at[0], gas[0])
    cv = pltpu.async_copy(vid_hbm.at[idxg_v.at[0]], vbuf.at[0], gvs[0])
    for k in range(NCH):
        cur, nxt = k % 2, (k + 1) % 2
        if k + 1 < NCH:
            if k >= 1:
                sc_a[k - 1].wait()
                sc_v[k - 1].wait()
            ca_n = pltpu.async_copy(aud_hbm.at[idxg_v.at[k + 1]],
                                    abuf.at[nxt], gas[nxt])
            cv_n = pltpu.async_copy(vid_hbm.at[idxg_v.at[k + 1]],
                                    vbuf.at[nxt], gvs[nxt])
        ca.wait()
        sc_a[k] = pltpu.async_copy(abuf.at[cur], aout.at[oidx_v.at[k]],
                                   sas[cur])
        cv.wait()
        sc_v[k] = pltpu.async_copy(vbuf.at[cur], vout.at[oidx_v.at[k]],
                                   svs[cur])
        if k + 1 < NCH:
            ca, cv = ca_n, cv_n
    sc_a[NCH - 2].wait()
    sc_v[NCH - 2].wait()
    sc_a[NCH - 1].wait()
    sc_v[NCH - 1].wait()

  return _sc_select_gather


def kernel(audio_sequence, video_sequence, text_global, Wa, ba, Wv, bv, Wt, bt):
    bav = ((ba + bv) * 0.5).reshape(1, HIDDEN)
    bt2 = bt.reshape(1, HIDDEN)
    text3 = text_global.reshape(B, 1, TEXT_DIM)

    scores3 = _scores_call(audio_sequence, video_sequence, text3,
                           Wa, Wv, Wt, bav, bt2)
    scores = scores3.reshape(B, S)
    rank3, pos3 = _rank_call(scores.reshape(B, 1, S), scores.T)

    aout, vout, iout = _get_sc_kernel()(
        rank3,
        pos3,
        audio_sequence.reshape(B * S, AUDIO_DIM),
        video_sequence.reshape(B * S, VIDEO_DIM),
    )

    selected_audio = jnp.transpose(
        aout.reshape(NSEL, B, AUDIO_DIM), (1, 0, 2))
    selected_video = jnp.transpose(
        vout.reshape(NSEL, B, VIDEO_DIM), (1, 0, 2))
    selected_indices = iout.reshape(B, IPAD)[:, :NSEL]
    mask = jnp.ones((B, NSEL), jnp.float32)
    return selected_audio, selected_video, selected_indices, mask
